# baseline (speedup 1.0000x reference)
import jax
import jax.numpy as jnp
from jax.experimental import pallas as pl

N = 207
HID = 64
T = 12
D_IN = 2
NCLS = 5
BS = 1


def _gconv(S, inp, st, Wmi, Wms):
    Y0 = inp @ Wmi[0] + st @ Wms[0]
    Y1 = inp @ Wmi[1] + st @ Wms[1]
    Y2 = inp @ Wmi[2] + st @ Wms[2]
    U = S @ Y2
    Z = S @ (Y1 + 2.0 * U)
    return Y0 - Y2 + Z


def _cell(S, inp, st, Wgi, Wgs, bg, Wci, Wcs, bc):
    val = jax.nn.sigmoid(_gconv(S, inp, st, Wgi, Wgs) + bg)
    r = val[:, :HID]
    u = val[:, HID:]
    c = jnp.tanh(_gconv(S, inp, r * st, Wci, Wcs) + bc)
    return u * st + (1.0 - u) * c


def _dcrnn_kernel(inp_ref, seq_ref, s_ref,
                  wg0i_ref, wg0s_ref, bg0_ref, wc0i_ref, wc0s_ref, bc0_ref,
                  wg1i_ref, wg1s_ref, bg1_ref, wc1i_ref, wc1s_ref, bc1_ref,
                  wfc_ref, bfc_ref, out_ref):
    S = s_ref[...]
    Wg0i = [wg0i_ref[m] for m in range(3)]
    Wg0s = [wg0s_ref[m] for m in range(3)]
    Wc0i = [wc0i_ref[m] for m in range(3)]
    Wc0s = [wc0s_ref[m] for m in range(3)]
    Wg1i = [wg1i_ref[m] for m in range(3)]
    Wg1s = [wg1s_ref[m] for m in range(3)]
    Wc1i = [wc1i_ref[m] for m in range(3)]
    Wc1s = [wc1s_ref[m] for m in range(3)]
    bg0 = bg0_ref[...]
    bc0 = bc0_ref[...]
    bg1 = bg1_ref[...]
    bc1 = bc1_ref[...]

    def step(t, carry):
        st0, st1, last = carry
        new0, new1, newl = [], [], []
        for b in range(BS):
            xt = inp_ref[b, pl.ds(t, 1)][0]
            s0 = _cell(S, xt, st0[b], Wg0i, Wg0s, bg0, Wc0i, Wc0s, bc0)
            s1 = _cell(S, s0, st1[b], Wg1i, Wg1s, bg1, Wc1i, Wc1s, bc1)
            L = seq_ref[b, 0, 0]
            new0.append(s0)
            new1.append(s1)
            newl.append(jnp.where(L == t + 1, s1, last[b]))
        return tuple(new0), tuple(new1), tuple(newl)

    z = tuple(jnp.zeros((N, HID), jnp.float32) for _ in range(BS))
    _, _, last = jax.lax.fori_loop(0, T, step, (z, z, z))

    for b in range(BS):
        logits = jax.nn.relu(last[b]) @ wfc_ref[...] + bfc_ref[...]
        out_ref[b, 0, :] = jnp.max(logits, axis=0)


def _split_w(W, d_in):
    isz = W.shape[0] // 3
    O = W.shape[1]
    Wm = jnp.transpose(W.reshape(isz, 3, O), (1, 0, 2))
    return Wm[:, :d_in, :], Wm[:, d_in:, :]


@jax.jit
def kernel(input_seq, seq_lengths, supports, Wg0, bg0, Wc0, bc0,
           Wg1, bg1, Wc1, bc1, Wfc, bfc):
    B = input_seq.shape[0]
    S = supports[0]
    Wg0i, Wg0s = _split_w(Wg0, D_IN)
    Wc0i, Wc0s = _split_w(Wc0, D_IN)
    Wg1i, Wg1s = _split_w(Wg1, HID)
    Wc1i, Wc1s = _split_w(Wc1, HID)
    seq = seq_lengths.astype(jnp.int32).reshape(B, 1, 1)

    def c(shape):
        return pl.BlockSpec(shape, lambda g: (0,) * len(shape))

    grid_spec = pl.GridSpec(
        grid=(B // BS,),
        in_specs=[
            pl.BlockSpec((BS, T, N, D_IN), lambda g: (g, 0, 0, 0)),
            pl.BlockSpec((BS, 1, 1), lambda g: (g, 0, 0)),
            c((N, N)),
            c(Wg0i.shape), c(Wg0s.shape), c((1, 2 * HID)),
            c(Wc0i.shape), c(Wc0s.shape), c((1, HID)),
            c(Wg1i.shape), c(Wg1s.shape), c((1, 2 * HID)),
            c(Wc1i.shape), c(Wc1s.shape), c((1, HID)),
            c((HID, NCLS)), c((1, NCLS)),
        ],
        out_specs=pl.BlockSpec((BS, 1, NCLS), lambda g: (g, 0, 0)),
    )
    out = pl.pallas_call(
        _dcrnn_kernel,
        grid_spec=grid_spec,
        out_shape=jax.ShapeDtypeStruct((B, 1, NCLS), jnp.float32),
    )(input_seq, seq, S,
      Wg0i, Wg0s, bg0.reshape(1, -1), Wc0i, Wc0s, bc0.reshape(1, -1),
      Wg1i, Wg1s, bg1.reshape(1, -1), Wc1i, Wc1s, bc1.reshape(1, -1),
      Wfc, bfc.reshape(1, -1))
    return out.reshape(B, NCLS)


# speedup vs baseline: 1.0244x; 1.0244x over previous
import jax
import jax.numpy as jnp
from jax.experimental import pallas as pl

N = 207
HID = 64
T = 12
D_IN = 2
NCLS = 5
BS = 1


BF = jnp.bfloat16


def _mm(a, b):
    return jnp.dot(a.astype(BF), b, preferred_element_type=jnp.float32)


def _gconv(S, inp, st, Wmi, Wms):
    Y0 = _mm(inp, Wmi[0]) + _mm(st, Wms[0])
    Y1 = _mm(inp, Wmi[1]) + _mm(st, Wms[1])
    Y2 = _mm(inp, Wmi[2]) + _mm(st, Wms[2])
    U = _mm(S, Y2.astype(BF))
    Z = _mm(S, (Y1 + 2.0 * U).astype(BF))
    return Y0 - Y2 + Z


def _cell(S, inp, st, Wgi, Wgs, bg, Wci, Wcs, bc):
    val = jax.nn.sigmoid(_gconv(S, inp, st, Wgi, Wgs) + bg)
    r = val[:, :HID]
    u = val[:, HID:]
    c = jnp.tanh(_gconv(S, inp, r * st, Wci, Wcs) + bc)
    return u * st + (1.0 - u) * c


def _dcrnn_kernel(inp_ref, seq_ref, s_ref,
                  wg0i_ref, wg0s_ref, bg0_ref, wc0i_ref, wc0s_ref, bc0_ref,
                  wg1i_ref, wg1s_ref, bg1_ref, wc1i_ref, wc1s_ref, bc1_ref,
                  wfc_ref, bfc_ref, out_ref):
    S = s_ref[...]
    Wg0i = [wg0i_ref[m] for m in range(3)]
    Wg0s = [wg0s_ref[m] for m in range(3)]
    Wc0i = [wc0i_ref[m] for m in range(3)]
    Wc0s = [wc0s_ref[m] for m in range(3)]
    Wg1i = [wg1i_ref[m] for m in range(3)]
    Wg1s = [wg1s_ref[m] for m in range(3)]
    Wc1i = [wc1i_ref[m] for m in range(3)]
    Wc1s = [wc1s_ref[m] for m in range(3)]
    bg0 = bg0_ref[...]
    bc0 = bc0_ref[...]
    bg1 = bg1_ref[...]
    bc1 = bc1_ref[...]

    st0 = [jnp.zeros((N, HID), jnp.float32) for _ in range(BS)]
    st1 = [jnp.zeros((N, HID), jnp.float32) for _ in range(BS)]
    last = [jnp.zeros((N, HID), jnp.float32) for _ in range(BS)]

    for t in range(T):
        for b in range(BS):
            xt = inp_ref[b, t]
            st0[b] = _cell(S, xt, st0[b], Wg0i, Wg0s, bg0, Wc0i, Wc0s, bc0)
            st1[b] = _cell(S, st0[b], st1[b], Wg1i, Wg1s, bg1, Wc1i, Wc1s, bc1)
            L = seq_ref[b, 0, 0]
            last[b] = jnp.where(L == t + 1, st1[b], last[b])

    for b in range(BS):
        logits = _mm(jax.nn.relu(last[b]), wfc_ref[...]) + bfc_ref[...]
        out_ref[b, 0, :] = jnp.max(logits, axis=0)


def _split_w(W, d_in):
    isz = W.shape[0] // 3
    O = W.shape[1]
    Wm = jnp.transpose(W.reshape(isz, 3, O), (1, 0, 2)).astype(BF)
    return Wm[:, :d_in, :], Wm[:, d_in:, :]


@jax.jit
def kernel(input_seq, seq_lengths, supports, Wg0, bg0, Wc0, bc0,
           Wg1, bg1, Wc1, bc1, Wfc, bfc):
    B = input_seq.shape[0]
    S = supports[0].astype(BF)
    inp = input_seq.astype(BF)
    Wg0i, Wg0s = _split_w(Wg0, D_IN)
    Wc0i, Wc0s = _split_w(Wc0, D_IN)
    Wg1i, Wg1s = _split_w(Wg1, HID)
    Wc1i, Wc1s = _split_w(Wc1, HID)
    Wfcb = Wfc.astype(BF)
    seq = seq_lengths.astype(jnp.int32).reshape(B, 1, 1)

    def c(shape):
        return pl.BlockSpec(shape, lambda g: (0,) * len(shape))

    grid_spec = pl.GridSpec(
        grid=(B // BS,),
        in_specs=[
            pl.BlockSpec((BS, T, N, D_IN), lambda g: (g, 0, 0, 0)),
            pl.BlockSpec((BS, 1, 1), lambda g: (g, 0, 0)),
            c((N, N)),
            c(Wg0i.shape), c(Wg0s.shape), c((1, 2 * HID)),
            c(Wc0i.shape), c(Wc0s.shape), c((1, HID)),
            c(Wg1i.shape), c(Wg1s.shape), c((1, 2 * HID)),
            c(Wc1i.shape), c(Wc1s.shape), c((1, HID)),
            c((HID, NCLS)), c((1, NCLS)),
        ],
        out_specs=pl.BlockSpec((BS, 1, NCLS), lambda g: (g, 0, 0)),
    )
    out = pl.pallas_call(
        _dcrnn_kernel,
        grid_spec=grid_spec,
        out_shape=jax.ShapeDtypeStruct((B, 1, NCLS), jnp.float32),
    )(inp, seq, S,
      Wg0i, Wg0s, bg0.reshape(1, -1), Wc0i, Wc0s, bc0.reshape(1, -1),
      Wg1i, Wg1s, bg1.reshape(1, -1), Wc1i, Wc1s, bc1.reshape(1, -1),
      Wfcb, bfc.reshape(1, -1))
    return out.reshape(B, NCLS)


# speedup vs baseline: 1.2074x; 1.1787x over previous
import jax
import jax.numpy as jnp
from jax.experimental import pallas as pl

N = 207
HID = 64
T = 12
D_IN = 2
NCLS = 5
BS = 1


def _gconv(S, inp, st, Wmi, Wms):
    Y0 = inp @ Wmi[0] + st @ Wms[0]
    Y1 = inp @ Wmi[1] + st @ Wms[1]
    Y2 = inp @ Wmi[2] + st @ Wms[2]
    U = S @ Y2
    Z = S @ (Y1 + 2.0 * U)
    return Y0 - Y2 + Z


def _cell(S, inp, st, Wgi, Wgs, bg, Wci, Wcs, bc):
    val = jax.nn.sigmoid(_gconv(S, inp, st, Wgi, Wgs) + bg)
    r = val[:, :HID]
    u = val[:, HID:]
    c = jnp.tanh(_gconv(S, inp, r * st, Wci, Wcs) + bc)
    return u * st + (1.0 - u) * c


def _dcrnn_kernel(inp_ref, seq_ref, s_ref,
                  wg0i_ref, wg0s_ref, bg0_ref, wc0i_ref, wc0s_ref, bc0_ref,
                  wg1i_ref, wg1s_ref, bg1_ref, wc1i_ref, wc1s_ref, bc1_ref,
                  wfc_ref, bfc_ref, out_ref):
    S = s_ref[...]
    Wg0i = [wg0i_ref[m] for m in range(3)]
    Wg0s = [wg0s_ref[m] for m in range(3)]
    Wc0i = [wc0i_ref[m] for m in range(3)]
    Wc0s = [wc0s_ref[m] for m in range(3)]
    Wg1i = [wg1i_ref[m] for m in range(3)]
    Wg1s = [wg1s_ref[m] for m in range(3)]
    Wc1i = [wc1i_ref[m] for m in range(3)]
    Wc1s = [wc1s_ref[m] for m in range(3)]
    bg0 = bg0_ref[...]
    bc0 = bc0_ref[...]
    bg1 = bg1_ref[...]
    bc1 = bc1_ref[...]

    st0 = [jnp.zeros((N, HID), jnp.float32) for _ in range(BS)]
    st1 = [jnp.zeros((N, HID), jnp.float32) for _ in range(BS)]
    last = [jnp.zeros((N, HID), jnp.float32) for _ in range(BS)]

    for t in range(T):
        for b in range(BS):
            xt = inp_ref[b, t]
            st0[b] = _cell(S, xt, st0[b], Wg0i, Wg0s, bg0, Wc0i, Wc0s, bc0)
            st1[b] = _cell(S, st0[b], st1[b], Wg1i, Wg1s, bg1, Wc1i, Wc1s, bc1)
            L = seq_ref[b, 0, 0]
            last[b] = jnp.where(L == t + 1, st1[b], last[b])

    for b in range(BS):
        logits = jax.nn.relu(last[b]) @ wfc_ref[...] + bfc_ref[...]
        out_ref[b, 0, :] = jnp.max(logits, axis=0)


def _split_w(W, d_in):
    isz = W.shape[0] // 3
    O = W.shape[1]
    Wm = jnp.transpose(W.reshape(isz, 3, O), (1, 0, 2))
    return Wm[:, :d_in, :], Wm[:, d_in:, :]


@jax.jit
def kernel(input_seq, seq_lengths, supports, Wg0, bg0, Wc0, bc0,
           Wg1, bg1, Wc1, bc1, Wfc, bfc):
    B = input_seq.shape[0]
    S = supports[0]
    inp = input_seq
    Wg0i, Wg0s = _split_w(Wg0, D_IN)
    Wc0i, Wc0s = _split_w(Wc0, D_IN)
    Wg1i, Wg1s = _split_w(Wg1, HID)
    Wc1i, Wc1s = _split_w(Wc1, HID)
    Wfcb = Wfc
    seq = seq_lengths.astype(jnp.int32).reshape(B, 1, 1)

    def c(shape):
        return pl.BlockSpec(shape, lambda g: (0,) * len(shape))

    grid_spec = pl.GridSpec(
        grid=(B // BS,),
        in_specs=[
            pl.BlockSpec((BS, T, N, D_IN), lambda g: (g, 0, 0, 0)),
            pl.BlockSpec((BS, 1, 1), lambda g: (g, 0, 0)),
            c((N, N)),
            c(Wg0i.shape), c(Wg0s.shape), c((1, 2 * HID)),
            c(Wc0i.shape), c(Wc0s.shape), c((1, HID)),
            c(Wg1i.shape), c(Wg1s.shape), c((1, 2 * HID)),
            c(Wc1i.shape), c(Wc1s.shape), c((1, HID)),
            c((HID, NCLS)), c((1, NCLS)),
        ],
        out_specs=pl.BlockSpec((BS, 1, NCLS), lambda g: (g, 0, 0)),
    )
    out = pl.pallas_call(
        _dcrnn_kernel,
        grid_spec=grid_spec,
        out_shape=jax.ShapeDtypeStruct((B, 1, NCLS), jnp.float32),
    )(inp, seq, S,
      Wg0i, Wg0s, bg0.reshape(1, -1), Wc0i, Wc0s, bc0.reshape(1, -1),
      Wg1i, Wg1s, bg1.reshape(1, -1), Wc1i, Wc1s, bc1.reshape(1, -1),
      Wfcb, bfc.reshape(1, -1))
    return out.reshape(B, NCLS)


# speedup vs baseline: 3.9636x; 3.2827x over previous
import jax
import jax.numpy as jnp
from jax.experimental import pallas as pl

N = 207
NP = 208
HID = 64
T = 12
D_IN = 2
NCLS = 5
G = 1


def _smat(S, X, O, npk):
    nb = X.shape[0] // NP
    outs = []
    for q in range(nb // npk):
        blk = jnp.concatenate(
            [X[(q * npk + j) * NP:(q * npk + j + 1) * NP] for j in range(npk)],
            axis=1)
        R = S @ blk
        outs.extend([R[:, j * O:(j + 1) * O] for j in range(npk)])
    return jnp.concatenate(outs, axis=0)


def _sdiff(S, Yall, O, npk):
    Y0 = Yall[:, :O]
    Y1 = Yall[:, O:2 * O]
    Y2 = Yall[:, 2 * O:]
    U = _smat(S, Y2, O, npk)
    Z = _smat(S, Y1 + 2.0 * U, O, npk)
    return Y0 - Y2 + Z


def _dcrnn_kernel(inp_ref, lrow_ref, s_ref,
                  wg0i_ref, wg0s_ref, bg0_ref, wc0i_ref, wc0s_ref, bc0_ref,
                  wi1_ref, wg1s_ref, bg1_ref, wc1s_ref, bc1_ref,
                  wfc_ref, bfc_ref, out_ref):
    SP = out_ref.shape[0]
    M = SP * NP
    S = s_ref[...]
    Wg0s = wg0s_ref[...]
    Wc0s = wc0s_ref[...]
    Wi1 = wi1_ref[...]
    Wg1s = wg1s_ref[...]
    Wc1s = wc1s_ref[...]
    bg0 = bg0_ref[...]
    bc0 = bc0_ref[...]
    bg1 = bg1_ref[...]
    bc1 = bc1_ref[...]
    lrow = lrow_ref[0]

    def step(t, carry):
        st0, st1, last = carry
        xtT = inp_ref[0, pl.ds(t, 1)][0]
        dn = (((0,), (0,)), ((), ()))
        ipg = jax.lax.dot_general(xtT, wg0i_ref[...], dn)
        ipc = jax.lax.dot_general(xtT, wc0i_ref[...], dn)
        Yg = ipg + st0 @ Wg0s
        val = jax.nn.sigmoid(_sdiff(S, Yg, 2 * HID, 2) + bg0)
        r = val[:, :HID]
        u = val[:, HID:]
        Yc = ipc + (r * st0) @ Wc0s
        c = jnp.tanh(_sdiff(S, Yc, HID, 4) + bc0)
        st0 = u * st0 + (1.0 - u) * c
        P = st0 @ Wi1
        Yg = P[:, :6 * HID] + st1 @ Wg1s
        val = jax.nn.sigmoid(_sdiff(S, Yg, 2 * HID, 2) + bg1)
        r = val[:, :HID]
        u = val[:, HID:]
        Yc = P[:, 6 * HID:] + (r * st1) @ Wc1s
        c = jnp.tanh(_sdiff(S, Yc, HID, 4) + bc1)
        st1 = u * st1 + (1.0 - u) * c
        last = jnp.where(lrow == t + 1, st1, last)
        return st0, st1, last

    z = jnp.zeros((M, HID), jnp.float32)
    _, _, last = jax.lax.fori_loop(0, T, step, (z, z, z))

    h = jax.nn.relu(last)
    logits = h @ wfc_ref[...] + bfc_ref[...]
    logits = jnp.where(lrow > 0, logits, -3.0e38)
    out_ref[:, 0, :] = jnp.max(logits.reshape(SP, NP, NCLS), axis=1)


def _deint(W, d_in):
    isz = W.shape[0] // 3
    O = W.shape[1]
    Wm = jnp.transpose(W.reshape(isz, 3, O), (1, 0, 2))
    Wcat = jnp.concatenate([Wm[0], Wm[1], Wm[2]], axis=1)
    return Wcat[:d_in], Wcat[d_in:]


@jax.jit
def kernel(input_seq, seq_lengths, supports, Wg0, bg0, Wc0, bc0,
           Wg1, bg1, Wc1, bc1, Wfc, bfc):
    B = input_seq.shape[0]
    SP = B // G
    M = SP * NP
    S = jnp.pad(supports[0], ((0, NP - N), (0, NP - N)))
    inp = jnp.pad(input_seq, ((0, 0), (0, 0), (0, NP - N), (0, 0)))
    inp = inp.reshape(G, SP, T, NP, D_IN).transpose(0, 2, 4, 1, 3)
    inp = inp.reshape(G, T, D_IN, M)
    lrow = jnp.repeat(seq_lengths.astype(jnp.int32), NP).reshape(B, NP)
    lrow = jnp.where(jnp.arange(NP) < N, lrow, 0).reshape(G, M, 1)
    Wg0i, Wg0s = _deint(Wg0, D_IN)
    Wc0i, Wc0s = _deint(Wc0, D_IN)
    Wg1i, Wg1s = _deint(Wg1, HID)
    Wc1i, Wc1s = _deint(Wc1, HID)
    Wi1 = jnp.concatenate([Wg1i, Wc1i], axis=1)

    def c(shape):
        return pl.BlockSpec(shape, lambda g: (0,) * len(shape))

    grid_spec = pl.GridSpec(
        grid=(G,),
        in_specs=[
            pl.BlockSpec((1, T, D_IN, M), lambda g: (g, 0, 0, 0)),
            pl.BlockSpec((1, M, 1), lambda g: (g, 0, 0)),
            c((NP, NP)),
            c(Wg0i.shape), c(Wg0s.shape), c((1, 2 * HID)),
            c(Wc0i.shape), c(Wc0s.shape), c((1, HID)),
            c(Wi1.shape), c(Wg1s.shape), c((1, 2 * HID)),
            c(Wc1s.shape), c((1, HID)),
            c((HID, NCLS)), c((1, NCLS)),
        ],
        out_specs=pl.BlockSpec((SP, 1, NCLS), lambda g: (g, 0, 0)),
    )
    out = pl.pallas_call(
        _dcrnn_kernel,
        grid_spec=grid_spec,
        out_shape=jax.ShapeDtypeStruct((B, 1, NCLS), jnp.float32),
    )(inp, lrow, S,
      Wg0i, Wg0s, bg0.reshape(1, -1), Wc0i, Wc0s, bc0.reshape(1, -1),
      Wi1, Wg1s, bg1.reshape(1, -1), Wc1s, bc1.reshape(1, -1),
      Wfc, bfc.reshape(1, -1))
    return out.reshape(B, NCLS)
